# Initial kernel scaffold; baseline (speedup 1.0000x reference)
import jax
import jax.numpy as jnp
from jax.experimental import pallas as pl


def kernel(x, edge_index, batch, W0, b0, W1, b1, W2, b2, Wc1, bc1, Wc2, bc2):
    raise NotImplementedError("write your pallas kernel here")


# speedup vs baseline: 16.0539x; 16.0539x over previous
import functools

import jax
import jax.numpy as jnp
from jax import lax
from jax.experimental import pallas as pl
from jax.experimental.pallas import tpu as pltpu
from jax.experimental.pallas import tpu_sc as plsc

N = 10000
E = 320000
D = 128
H = 128
C = 2
G = 64

NC = 2
NS = 16
NW = NC * NS
EK = 80
ECHUNKS = E // (NW * EK)
NP = 10240
ROWS_PER_TILE = NP // NS
ZROWS = 128

BN = 1000
NB = N // BN

_f32 = jnp.float32


def _zero_vmem_rows(buf, nrows, width):
    zv = jnp.zeros((16,), _f32)

    def body(r, _):
        for f in range(width // 16):
            buf[r, pl.ds(f * 16, 16)] = zv
        return 0

    lax.fori_loop(0, nrows, body, 0)


def _deg_body(dst_hbm, out, dstb, ones, s_sh, sem):
    c = lax.axis_index("c")
    s = lax.axis_index("s")
    w = c * NS + s
    base = s * ROWS_PER_TILE

    pltpu.sync_copy(dst_hbm.at[w], dstb)

    _zero_vmem_rows(ones, EK, H)
    for q in range(ROWS_PER_TILE // EK):
        pltpu.sync_copy(ones, s_sh.at[pl.ds(base + q * EK, EK)])
    plsc.subcore_barrier()

    ov = jnp.ones((16,), _f32)

    def fill_ones(r, _):
        for f in range(H // 16):
            ones[r, pl.ds(f * 16, 16)] = ov
        return 0

    lax.fori_loop(0, EK, fill_ones, 0)

    def chunk(j, _):
        pltpu.sync_copy(ones, s_sh.at[dstb.at[j]], add=True)
        return 0

    lax.fori_loop(0, ECHUNKS, chunk, 0)
    plsc.subcore_barrier()

    sl = pl.ds(base, ROWS_PER_TILE)
    pltpu.sync_copy(s_sh.at[sl], out.at[c, sl])


_deg_call = functools.partial(
    pl.kernel,
    mesh=plsc.VectorSubcoreMesh(core_axis_name="c", subcore_axis_name="s"),
    out_type=jax.ShapeDtypeStruct((NC, NP, H), _f32),
    scratch_types=[
        pltpu.VMEM((ECHUNKS, EK), jnp.int32),
        pltpu.VMEM((EK, H), _f32),
        pltpu.VMEM_SHARED((NP, H), _f32),
        pltpu.SemaphoreType.DMA,
    ],
)(_deg_body)


def _scatter_body(hp_hbm, src_hbm, dst_hbm, out,
                  srcb, dstb, rows, s_sh, sem):
    c = lax.axis_index("c")
    s = lax.axis_index("s")
    w = c * NS + s
    base = s * ROWS_PER_TILE

    pltpu.sync_copy(src_hbm.at[w], srcb)
    pltpu.sync_copy(dst_hbm.at[w], dstb)

    _zero_vmem_rows(rows, EK, H)
    for q in range(ROWS_PER_TILE // EK):
        pltpu.sync_copy(rows, s_sh.at[pl.ds(base + q * EK, EK)])
    plsc.subcore_barrier()

    def chunk(j, _):
        pltpu.async_copy(hp_hbm.at[srcb.at[j]], rows, sem).wait()
        pltpu.sync_copy(rows, s_sh.at[dstb.at[j]], add=True)
        return 0

    lax.fori_loop(0, ECHUNKS, chunk, 0)
    plsc.subcore_barrier()

    sl = pl.ds(base, ROWS_PER_TILE)
    pltpu.sync_copy(s_sh.at[sl], out.at[c, sl])


_scatter_call = functools.partial(
    pl.kernel,
    mesh=plsc.VectorSubcoreMesh(core_axis_name="c", subcore_axis_name="s"),
    out_type=jax.ShapeDtypeStruct((NC, NP, H), _f32),
    scratch_types=[
        pltpu.VMEM((ECHUNKS, EK), jnp.int32),
        pltpu.VMEM((ECHUNKS, EK), jnp.int32),
        pltpu.VMEM((EK, H), _f32),
        pltpu.VMEM_SHARED((NP, H), _f32),
        pltpu.SemaphoreType.DMA,
    ],
)(_scatter_body)


def _prep_body(x_ref, w_ref, d0_ref, d1_ref, dinv_ref, hp_ref):
    deg = d0_ref[0, :, 0:1] + d1_ref[0, :, 0:1] + 1.0
    dinv = lax.rsqrt(deg)
    dinv_ref[...] = dinv
    hp_ref[...] = dinv * jnp.dot(x_ref[...], w_ref[...],
                                 preferred_element_type=_f32)


def _prep_call(x, W, degp):
    return pl.pallas_call(
        _prep_body,
        grid=(NB,),
        in_specs=[
            pl.BlockSpec((BN, D), lambda i: (i, 0)),
            pl.BlockSpec((D, H), lambda i: (0, 0)),
            pl.BlockSpec((1, BN, 128), lambda i: (0, i, 0)),
            pl.BlockSpec((1, BN, 128), lambda i: (1, i, 0)),
        ],
        out_specs=[
            pl.BlockSpec((BN, 1), lambda i: (i, 0)),
            pl.BlockSpec((BN, H), lambda i: (i, 0)),
        ],
        out_shape=[
            jax.ShapeDtypeStruct((N, 1), _f32),
            jax.ShapeDtypeStruct((N, H), _f32),
        ],
    )(x, W, degp, degp)


def _mid_body(s0_ref, s1_ref, hp_ref, dinv_ref, b_ref, w_ref, out_ref):
    dinv = dinv_ref[...]
    h = dinv * (s0_ref[0] + s1_ref[0] + hp_ref[...]) + b_ref[...]
    h = jnp.maximum(h, 0.0)
    out_ref[...] = dinv * jnp.dot(h, w_ref[...], preferred_element_type=_f32)


def _mid_call(S, hp, dinv, b, W):
    return pl.pallas_call(
        _mid_body,
        grid=(NB,),
        in_specs=[
            pl.BlockSpec((1, BN, H), lambda i: (0, i, 0)),
            pl.BlockSpec((1, BN, H), lambda i: (1, i, 0)),
            pl.BlockSpec((BN, H), lambda i: (i, 0)),
            pl.BlockSpec((BN, 1), lambda i: (i, 0)),
            pl.BlockSpec((1, H), lambda i: (0, 0)),
            pl.BlockSpec((H, H), lambda i: (0, 0)),
        ],
        out_specs=pl.BlockSpec((BN, H), lambda i: (i, 0)),
        out_shape=jax.ShapeDtypeStruct((N, H), _f32),
    )(S, S, hp, dinv, b, W)


def _final_body(s0_ref, s1_ref, hp_ref, dinv_ref, b_ref, batch_ref,
                wc1_ref, bc1_ref, wc2_ref, bc2_ref, out_ref, acc, cnt):
    i = pl.program_id(0)

    @pl.when(i == 0)
    def _():
        acc[...] = jnp.zeros((G, H), _f32)
        cnt[...] = jnp.zeros((G, H), _f32)

    dinv = dinv_ref[...]
    agg = dinv * (s0_ref[0] + s1_ref[0] + hp_ref[...]) + b_ref[...]
    bb = batch_ref[0, 0, :]
    gids = lax.broadcasted_iota(jnp.int32, (G, BN), 0)
    onehot = (bb[None, :] == gids).astype(_f32)
    acc[...] += jnp.dot(onehot, agg, preferred_element_type=_f32)
    cnt[...] += jnp.broadcast_to(jnp.sum(onehot, axis=1, keepdims=True), (G, H))

    @pl.when(i == NB - 1)
    def _():
        pooled = acc[...] / jnp.maximum(cnt[...], 1.0)
        z = jnp.maximum(
            jnp.dot(pooled, wc1_ref[...], preferred_element_type=_f32)
            + bc1_ref[...], 0.0)
        out_ref[...] = (jnp.dot(z, wc2_ref[...], preferred_element_type=_f32)
                        + bc2_ref[...])


def _final_call(S, hp, dinv, b, batch3, Wc1p, bc1p, Wc2p, bc2p):
    return pl.pallas_call(
        _final_body,
        grid=(NB,),
        in_specs=[
            pl.BlockSpec((1, BN, H), lambda i: (0, i, 0)),
            pl.BlockSpec((1, BN, H), lambda i: (1, i, 0)),
            pl.BlockSpec((BN, H), lambda i: (i, 0)),
            pl.BlockSpec((BN, 1), lambda i: (i, 0)),
            pl.BlockSpec((1, H), lambda i: (0, 0)),
            pl.BlockSpec((1, 1, BN), lambda i: (i, 0, 0)),
            pl.BlockSpec((H, H), lambda i: (0, 0)),
            pl.BlockSpec((1, H), lambda i: (0, 0)),
            pl.BlockSpec((H, H), lambda i: (0, 0)),
            pl.BlockSpec((1, H), lambda i: (0, 0)),
        ],
        out_specs=pl.BlockSpec((G, H), lambda i: (0, 0)),
        out_shape=jax.ShapeDtypeStruct((G, H), _f32),
        scratch_shapes=[
            pltpu.VMEM((G, H), _f32),
            pltpu.VMEM((G, H), _f32),
        ],
    )(S, S, hp, dinv, b, batch3, Wc1p, bc1p, Wc2p, bc2p)


def kernel(x, edge_index, batch, W0, b0, W1, b1, W2, b2, Wc1, bc1, Wc2, bc2):
    src2d = edge_index[0].reshape(NW, ECHUNKS, EK)
    dst2d = edge_index[1].reshape(NW, ECHUNKS, EK)
    batch3 = batch.reshape(NB, 1, BN)

    Wc1p = jnp.zeros((H, H), _f32).at[:, : H // 2].set(Wc1)
    bc1p = jnp.zeros((1, H), _f32).at[0, : H // 2].set(bc1)
    Wc2p = jnp.zeros((H, H), _f32).at[: H // 2, :C].set(Wc2)
    bc2p = jnp.zeros((1, H), _f32).at[0, :C].set(bc2)

    degp = _deg_call(dst2d)
    dinv, hp0 = _prep_call(x, W0, degp)
    S0 = _scatter_call(hp0, src2d, dst2d)
    hp1 = _mid_call(S0, hp0, dinv, b0.reshape(1, H), W1)
    S1 = _scatter_call(hp1, src2d, dst2d)
    hp2 = _mid_call(S1, hp1, dinv, b1.reshape(1, H), W2)
    S2 = _scatter_call(hp2, src2d, dst2d)
    out = _final_call(S2, hp2, dinv, b2.reshape(1, H), batch3,
                      Wc1p, bc1p, Wc2p, bc2p)
    return out[:, :C]


# speedup vs baseline: 18.6678x; 1.1628x over previous
import functools

import jax
import jax.numpy as jnp
from jax import lax
from jax.experimental import pallas as pl
from jax.experimental.pallas import tpu as pltpu
from jax.experimental.pallas import tpu_sc as plsc

N = 10000
E = 320000
D = 128
H = 128
C = 2
G = 64

NC = 2
NS = 16
NW = NC * NS
EK = 40
ECHUNKS = E // (NW * EK)
NP = 10240
ROWS_PER_TILE = NP // NS
ZROWS = 128

BN = 1000
NB = N // BN

_f32 = jnp.float32


def _zero_vmem_rows(buf, nrows, width):
    zv = jnp.zeros((16,), _f32)

    def body(r, _):
        for f in range(width // 16):
            buf[r, pl.ds(f * 16, 16)] = zv
        return 0

    lax.fori_loop(0, nrows, body, 0)


def _deg_body(dst_hbm, out, dstb, ones, s_sh, sem):
    c = lax.axis_index("c")
    s = lax.axis_index("s")
    w = c * NS + s
    base = s * ROWS_PER_TILE

    pltpu.sync_copy(dst_hbm.at[w], dstb)

    _zero_vmem_rows(ones, EK, H)
    for q in range(ROWS_PER_TILE // EK):
        pltpu.sync_copy(ones, s_sh.at[pl.ds(base + q * EK, EK)])
    plsc.subcore_barrier()

    ov = jnp.ones((16,), _f32)

    def fill_ones(r, _):
        for f in range(H // 16):
            ones[r, pl.ds(f * 16, 16)] = ov
        return 0

    lax.fori_loop(0, EK, fill_ones, 0)

    def chunk(j, _):
        pltpu.sync_copy(ones, s_sh.at[dstb.at[j]], add=True)
        return 0

    lax.fori_loop(0, ECHUNKS, chunk, 0)
    plsc.subcore_barrier()

    sl = pl.ds(base, ROWS_PER_TILE)
    pltpu.sync_copy(s_sh.at[sl], out.at[c, sl])


_deg_call = functools.partial(
    pl.kernel,
    mesh=plsc.VectorSubcoreMesh(core_axis_name="c", subcore_axis_name="s"),
    out_type=jax.ShapeDtypeStruct((NC, NP, H), _f32),
    scratch_types=[
        pltpu.VMEM((ECHUNKS, EK), jnp.int32),
        pltpu.VMEM((EK, H), _f32),
        pltpu.VMEM_SHARED((NP, H), _f32),
        pltpu.SemaphoreType.DMA,
    ],
)(_deg_body)


NBUF = 5
SBC = 25
NSB = ECHUNKS // SBC
SBGRP = SBC // NBUF


def _scatter_body(hp_hbm, src_hbm, dst_hbm, out,
                  srcb, dstb, r0, r1, r2, r3, r4, s_sh, gsem, ssem):
    c = lax.axis_index("c")
    s = lax.axis_index("s")
    w = c * NS + s
    base = s * ROWS_PER_TILE
    bufs = (r0, r1, r2, r3, r4)

    def drain_adds():
        for b in bufs:
            pltpu.make_async_copy(b, s_sh.at[pl.ds(0, EK)], ssem).wait()

    _zero_vmem_rows(r0, EK, H)
    for q in range(ROWS_PER_TILE // EK):
        pltpu.sync_copy(r0, s_sh.at[pl.ds(base + q * EK, EK)])
    plsc.subcore_barrier()

    def superblock(sb, _):
        @pl.when(sb > 0)
        def _():
            drain_adds()

        pltpu.sync_copy(src_hbm.at[w, sb], srcb)
        pltpu.sync_copy(dst_hbm.at[w, sb], dstb)

        def group(g, _):
            @pl.when(g > 0)
            def _():
                drain_adds()

            hs = [pltpu.async_copy(hp_hbm.at[srcb.at[g * NBUF + i]],
                                   bufs[i], gsem)
                  for i in range(NBUF)]
            for h in hs:
                h.wait()
            for i in range(NBUF):
                pltpu.async_copy(bufs[i], s_sh.at[dstb.at[g * NBUF + i]],
                                 ssem, add=True)
            return 0

        lax.fori_loop(0, SBGRP, group, 0)
        return 0

    lax.fori_loop(0, NSB, superblock, 0)
    drain_adds()
    plsc.subcore_barrier()

    sl = pl.ds(base, ROWS_PER_TILE)
    pltpu.sync_copy(s_sh.at[sl], out.at[c, sl])


_scatter_call = functools.partial(
    pl.kernel,
    mesh=plsc.VectorSubcoreMesh(core_axis_name="c", subcore_axis_name="s"),
    out_type=jax.ShapeDtypeStruct((NC, NP, H), _f32),
    scratch_types=[
        pltpu.VMEM((SBC, EK), jnp.int32),
        pltpu.VMEM((SBC, EK), jnp.int32),
        pltpu.VMEM((EK, H), _f32),
        pltpu.VMEM((EK, H), _f32),
        pltpu.VMEM((EK, H), _f32),
        pltpu.VMEM((EK, H), _f32),
        pltpu.VMEM((EK, H), _f32),
        pltpu.VMEM_SHARED((NP, H), _f32),
        pltpu.SemaphoreType.DMA,
        pltpu.SemaphoreType.DMA,
    ],
)(_scatter_body)


def _prep_body(x_ref, w_ref, d0_ref, d1_ref, dinv_ref, hp_ref):
    deg = d0_ref[0, :, 0:1] + d1_ref[0, :, 0:1] + 1.0
    dinv = lax.rsqrt(deg)
    dinv_ref[...] = dinv
    hp_ref[...] = dinv * jnp.dot(x_ref[...], w_ref[...],
                                 preferred_element_type=_f32)


def _prep_call(x, W, degp):
    return pl.pallas_call(
        _prep_body,
        grid=(NB,),
        in_specs=[
            pl.BlockSpec((BN, D), lambda i: (i, 0)),
            pl.BlockSpec((D, H), lambda i: (0, 0)),
            pl.BlockSpec((1, BN, 128), lambda i: (0, i, 0)),
            pl.BlockSpec((1, BN, 128), lambda i: (1, i, 0)),
        ],
        out_specs=[
            pl.BlockSpec((BN, 1), lambda i: (i, 0)),
            pl.BlockSpec((BN, H), lambda i: (i, 0)),
        ],
        out_shape=[
            jax.ShapeDtypeStruct((N, 1), _f32),
            jax.ShapeDtypeStruct((N, H), _f32),
        ],
    )(x, W, degp, degp)


def _mid_body(s0_ref, s1_ref, hp_ref, dinv_ref, b_ref, w_ref, out_ref):
    dinv = dinv_ref[...]
    h = dinv * (s0_ref[0] + s1_ref[0] + hp_ref[...]) + b_ref[...]
    h = jnp.maximum(h, 0.0)
    out_ref[...] = dinv * jnp.dot(h, w_ref[...], preferred_element_type=_f32)


def _mid_call(S, hp, dinv, b, W):
    return pl.pallas_call(
        _mid_body,
        grid=(NB,),
        in_specs=[
            pl.BlockSpec((1, BN, H), lambda i: (0, i, 0)),
            pl.BlockSpec((1, BN, H), lambda i: (1, i, 0)),
            pl.BlockSpec((BN, H), lambda i: (i, 0)),
            pl.BlockSpec((BN, 1), lambda i: (i, 0)),
            pl.BlockSpec((1, H), lambda i: (0, 0)),
            pl.BlockSpec((H, H), lambda i: (0, 0)),
        ],
        out_specs=pl.BlockSpec((BN, H), lambda i: (i, 0)),
        out_shape=jax.ShapeDtypeStruct((N, H), _f32),
    )(S, S, hp, dinv, b, W)


def _final_body(s0_ref, s1_ref, hp_ref, dinv_ref, b_ref, batch_ref,
                wc1_ref, bc1_ref, wc2_ref, bc2_ref, out_ref, acc, cnt):
    i = pl.program_id(0)

    @pl.when(i == 0)
    def _():
        acc[...] = jnp.zeros((G, H), _f32)
        cnt[...] = jnp.zeros((G, H), _f32)

    dinv = dinv_ref[...]
    agg = dinv * (s0_ref[0] + s1_ref[0] + hp_ref[...]) + b_ref[...]
    bb = batch_ref[0, 0, :]
    gids = lax.broadcasted_iota(jnp.int32, (G, BN), 0)
    onehot = (bb[None, :] == gids).astype(_f32)
    acc[...] += jnp.dot(onehot, agg, preferred_element_type=_f32)
    cnt[...] += jnp.broadcast_to(jnp.sum(onehot, axis=1, keepdims=True), (G, H))

    @pl.when(i == NB - 1)
    def _():
        pooled = acc[...] / jnp.maximum(cnt[...], 1.0)
        z = jnp.maximum(
            jnp.dot(pooled, wc1_ref[...], preferred_element_type=_f32)
            + bc1_ref[...], 0.0)
        out_ref[...] = (jnp.dot(z, wc2_ref[...], preferred_element_type=_f32)
                        + bc2_ref[...])


def _final_call(S, hp, dinv, b, batch3, Wc1p, bc1p, Wc2p, bc2p):
    return pl.pallas_call(
        _final_body,
        grid=(NB,),
        in_specs=[
            pl.BlockSpec((1, BN, H), lambda i: (0, i, 0)),
            pl.BlockSpec((1, BN, H), lambda i: (1, i, 0)),
            pl.BlockSpec((BN, H), lambda i: (i, 0)),
            pl.BlockSpec((BN, 1), lambda i: (i, 0)),
            pl.BlockSpec((1, H), lambda i: (0, 0)),
            pl.BlockSpec((1, 1, BN), lambda i: (i, 0, 0)),
            pl.BlockSpec((H, H), lambda i: (0, 0)),
            pl.BlockSpec((1, H), lambda i: (0, 0)),
            pl.BlockSpec((H, H), lambda i: (0, 0)),
            pl.BlockSpec((1, H), lambda i: (0, 0)),
        ],
        out_specs=pl.BlockSpec((G, H), lambda i: (0, 0)),
        out_shape=jax.ShapeDtypeStruct((G, H), _f32),
        scratch_shapes=[
            pltpu.VMEM((G, H), _f32),
            pltpu.VMEM((G, H), _f32),
        ],
    )(S, S, hp, dinv, b, batch3, Wc1p, bc1p, Wc2p, bc2p)


def kernel(x, edge_index, batch, W0, b0, W1, b1, W2, b2, Wc1, bc1, Wc2, bc2):
    src4d = edge_index[0].reshape(NW, NSB, SBC, EK)
    dst4d = edge_index[1].reshape(NW, NSB, SBC, EK)
    dst2d = edge_index[1].reshape(NW, ECHUNKS, EK)
    batch3 = batch.reshape(NB, 1, BN)

    Wc1p = jnp.zeros((H, H), _f32).at[:, : H // 2].set(Wc1)
    bc1p = jnp.zeros((1, H), _f32).at[0, : H // 2].set(bc1)
    Wc2p = jnp.zeros((H, H), _f32).at[: H // 2, :C].set(Wc2)
    bc2p = jnp.zeros((1, H), _f32).at[0, :C].set(bc2)

    degp = _deg_call(dst2d)
    dinv, hp0 = _prep_call(x, W0, degp)
    S0 = _scatter_call(hp0, src4d, dst4d)
    hp1 = _mid_call(S0, hp0, dinv, b0.reshape(1, H), W1)
    S1 = _scatter_call(hp1, src4d, dst4d)
    hp2 = _mid_call(S1, hp1, dinv, b1.reshape(1, H), W2)
    S2 = _scatter_call(hp2, src4d, dst4d)
    out = _final_call(S2, hp2, dinv, b2.reshape(1, H), batch3,
                      Wc1p, bc1p, Wc2p, bc2p)
    return out[:, :C]
